# baseline (device time: 186004 ns/iter reference)
import os as _os

import jax
import jax.numpy as jnp
from jax import lax
from jax.experimental import pallas as pl
from jax.experimental.pallas import tpu as pltpu

N_DEV = 4
M_LOC = 1024
K = 4096
N_TOT = 8192
N_BLK = N_TOT // N_DEV
W_KT = 256
W_SUB = 4
W_SLOTS = 3
K_DOT = 256
STAGES_PER_DOT = K_DOT // W_KT
DOTS_PER_BLK = K // K_DOT
K_TILES = K // W_KT
N_TILES = N_DEV * K_TILES

_RDMA = True
_DIAG = _os.environ.get("KDIAG", "")


def kernel(x, w_mat):
    def body(x_ref, w_ref, out_ref,
             xb, w_buf, wb, y4, q_send, q_recv, amax_mine, amax_all,
             w_sems, x_sems, o_sems,
             ax_send_sems, ax_recv_sems, d_send_sems, d_recv_sems):
        my = lax.axis_index("i")

        if _RDMA:
            bsem = pltpu.get_barrier_semaphore()
            for h in (1, 2, 3):
                pl.semaphore_signal(
                    bsem, inc=1,
                    device_id=((my + h) % N_DEV,),
                    device_id_type=pl.DeviceIdType.MESH,
                )
            pl.semaphore_wait(bsem, 3)

        amax_all[...] = jnp.zeros((N_DEV, 8, 128), jnp.float32)

        for half in (0, 1):
            pltpu.make_async_copy(
                x_ref.at[:, pl.ds(half * N_BLK, N_BLK)],
                y4.at[half], x_sems.at[half],
            ).start()
        for half in (0, 1):
            pltpu.make_async_copy(
                x_ref.at[:, pl.ds(half * N_BLK, N_BLK)],
                y4.at[half], x_sems.at[half],
            ).wait()

            def xstep(c, carry, half=half):
                sl = pl.ds(c * 256, 256)
                xb[sl, half * N_BLK:(half + 1) * N_BLK] = (
                    y4[half, sl, :].astype(jnp.bfloat16))
                return carry
            lax.fori_loop(0, M_LOC // 256, xstep, 0)

        def w_copies(col_blk, kk, slot):
            rows = W_KT // W_SUB
            return [
                pltpu.make_async_copy(
                    w_ref.at[pl.ds(kk * W_KT + q * rows, rows),
                             pl.ds(col_blk * N_BLK, N_BLK)],
                    w_buf.at[slot, pl.ds(q * rows, rows), :],
                    w_sems.at[slot, q],
                )
                for q in range(W_SUB)
            ]

        for c in w_copies(my, 0, 0):
            c.start()
        for c in w_copies(my, 1, 1):
            c.start()
        amax = jnp.float32(0.0)
        for h in range(N_DEV):
            jb_cur = (my + h) % N_DEV
            y4[h, :, :] = jnp.zeros((M_LOC, N_BLK), jnp.float32)

            def kt_step(t, carry, h=h, jb_cur=jb_cur):
                g = h * K_TILES + t
                slot = lax.rem(g, W_SLOTS)

                @pl.when(g + 2 < N_TILES)
                def _():
                    nxt = g + 2
                    cb = (my + nxt // K_TILES) % N_DEV
                    kk = lax.rem(nxt, K_TILES)
                    for c in w_copies(cb, kk, lax.rem(nxt, W_SLOTS)):
                        c.start()

                for c in w_copies(jb_cur, t, slot):
                    c.wait()
                wb[...] = w_buf[slot].astype(jnp.bfloat16)
                if _DIAG == "nomxu":
                    return carry
                part = lax.dot_general(
                    xb[:, pl.ds(t * W_KT, W_KT)], wb[...],
                    dimension_numbers=(((1,), (0,)), ((), ())),
                    preferred_element_type=jnp.float32,
                )
                if _DIAG == "noacc":
                    y4[h, :, :] = part
                else:
                    y4[h, :, :] += part
                return carry

            lax.fori_loop(0, K_TILES, kt_step, 0)
            if not _DIAG:
                amax = jnp.maximum(amax, jnp.max(jnp.abs(y4[h])))

        amax_mine[...] = jnp.full((8, 128), amax, jnp.float32)
        if _RDMA:
            for h in (1, 2, 3):
                pltpu.make_async_remote_copy(
                    src_ref=amax_mine,
                    dst_ref=amax_all.at[N_DEV - h],
                    send_sem=ax_send_sems.at[h],
                    recv_sem=ax_recv_sems.at[N_DEV - h],
                    device_id=((my + h) % N_DEV,),
                    device_id_type=pl.DeviceIdType.MESH,
                ).start()
            for r in (1, 2, 3):
                pltpu.make_async_remote_copy(
                    src_ref=amax_mine,
                    dst_ref=amax_all.at[r],
                    send_sem=ax_send_sems.at[0],
                    recv_sem=ax_recv_sems.at[r],
                    device_id=(my,),
                    device_id_type=pl.DeviceIdType.MESH,
                ).wait_recv()
        amax_g = jnp.maximum(amax, jnp.max(amax_all[...]))
        inv = 448.0 / amax_g
        scale = amax_g / 448.0

        def quant(v):
            return jnp.clip(v * inv, -448.0, 448.0).astype(jnp.float8_e4m3fn)

        R_CHUNK = 256
        N_CHUNKS = M_LOC // R_CHUNK

        for h in (1, 2, 3):
            def qstep(c, carry, h=h):
                sl = pl.ds(c * R_CHUNK, R_CHUNK)
                q_send[h - 1, sl, :] = quant(y4[h, sl, :])
                return carry
            lax.fori_loop(0, N_CHUNKS, qstep, 0)
            if _RDMA:
                pltpu.make_async_remote_copy(
                    src_ref=q_send.at[h - 1],
                    dst_ref=q_recv.at[3 - h],
                    send_sem=d_send_sems.at[h],
                    recv_sem=d_recv_sems.at[N_DEV - h],
                    device_id=((my + h) % N_DEV,),
                    device_id_type=pl.DeviceIdType.MESH,
                ).start()

        def out_store(s_idx, yslot):
            return pltpu.make_async_copy(
                y4.at[yslot],
                out_ref.at[pl.ds(s_idx * M_LOC, M_LOC), :],
                o_sems.at[yslot],
            )

        def own_step(c, carry):
            sl = pl.ds(c * R_CHUNK, R_CHUNK)
            y4[0, sl, :] = quant(y4[0, sl, :]).astype(jnp.float32) * scale
            return carry
        lax.fori_loop(0, N_CHUNKS, own_step, 0)
        out_store(my, 0).start()

        for n, r in enumerate((3, 2, 1)):
            if _RDMA:
                pltpu.make_async_remote_copy(
                    src_ref=q_send.at[0],
                    dst_ref=q_recv.at[r - 1],
                    send_sem=d_send_sems.at[0],
                    recv_sem=d_recv_sems.at[r],
                    device_id=(my,),
                    device_id_type=pl.DeviceIdType.MESH,
                ).wait_recv()
            src_ref = q_recv if _RDMA else q_send
            def dstep(c, carry, n=n, r=r, src_ref=src_ref):
                sl = pl.ds(c * R_CHUNK, R_CHUNK)
                y4[1 + n, sl, :] = (
                    src_ref[r - 1, sl, :].astype(jnp.float32) * scale)
                return carry
            lax.fori_loop(0, N_CHUNKS, dstep, 0)
            out_store((my + r) % N_DEV, 1 + n).start()

        for yslot in range(N_DEV):
            out_store(0, yslot).wait()

        for h in (1, 2, 3) if _RDMA else ():
            pltpu.make_async_remote_copy(
                src_ref=amax_mine,
                dst_ref=amax_all.at[N_DEV - h],
                send_sem=ax_send_sems.at[h],
                recv_sem=ax_recv_sems.at[0],
                device_id=((my + h) % N_DEV,),
                device_id_type=pl.DeviceIdType.MESH,
            ).wait_send()
            pltpu.make_async_remote_copy(
                src_ref=q_send.at[h - 1],
                dst_ref=q_recv.at[3 - h],
                send_sem=d_send_sems.at[h],
                recv_sem=d_recv_sems.at[0],
                device_id=((my + h) % N_DEV,),
                device_id_type=pl.DeviceIdType.MESH,
            ).wait_send()

    return pl.pallas_call(
        body,
        out_shape=jax.ShapeDtypeStruct((N_DEV * M_LOC, N_BLK), jnp.float32),
        in_specs=[
            pl.BlockSpec(memory_space=pltpu.HBM),
            pl.BlockSpec(memory_space=pltpu.HBM),
        ],
        out_specs=pl.BlockSpec(memory_space=pltpu.HBM),
        scratch_shapes=[
            pltpu.VMEM((M_LOC, K), jnp.bfloat16),
            pltpu.VMEM((W_SLOTS, W_KT, N_BLK), jnp.float32),
            pltpu.VMEM((W_KT, N_BLK), jnp.bfloat16),
            pltpu.VMEM((N_DEV, M_LOC, N_BLK), jnp.float32),
            pltpu.VMEM((3, M_LOC, N_BLK), jnp.float8_e4m3fn),
            pltpu.VMEM((3, M_LOC, N_BLK), jnp.float8_e4m3fn),
            pltpu.VMEM((8, 128), jnp.float32),
            pltpu.VMEM((N_DEV, 8, 128), jnp.float32),
            pltpu.SemaphoreType.DMA((W_SLOTS, W_SUB)),
            pltpu.SemaphoreType.DMA((2,)),
            pltpu.SemaphoreType.DMA((4,)),
            pltpu.SemaphoreType.DMA((4,)),
            pltpu.SemaphoreType.DMA((4,)),
            pltpu.SemaphoreType.DMA((4,)),
            pltpu.SemaphoreType.DMA((4,)),
        ],
        compiler_params=pltpu.CompilerParams(
            collective_id=0,
            vmem_limit_bytes=100 * 1024 * 1024,
            allow_collective_id_without_custom_barrier=not _RDMA,
        ),
    )(x, w_mat)
